# baseline (speedup 1.0000x reference)
import functools

import jax
import jax.numpy as jnp
from jax import lax
from jax.experimental import pallas as pl
from jax.experimental.pallas import tpu as pltpu
from jax.experimental.pallas import tpu_sc as plsc

NUM_SIGNALS = 100000
ROW_WIDTH = 70
BATCH = 4096

_info = plsc.get_sparse_core_info()
_NC, _NS = _info.num_cores, _info.num_subcores
_NW = _NC * _NS
_B_PER_W = BATCH // _NW


def _make_gather():
    mesh = plsc.VectorSubcoreMesh(core_axis_name="c", subcore_axis_name="s")

    @functools.partial(
        pl.kernel,
        mesh=mesh,
        out_type=jax.ShapeDtypeStruct((BATCH, ROW_WIDTH), jnp.float32),
        scratch_types=[
            pltpu.VMEM((_B_PER_W,), jnp.int32),
            pltpu.VMEM((_B_PER_W, ROW_WIDTH), jnp.float32),
            pltpu.SemaphoreType.DMA,
        ],
        compiler_params=pltpu.CompilerParams(use_tc_tiling_on_sc=False),
    )
    def gather_kernel(idx_hbm, table_hbm, out_hbm, idx_v, rows_v, sem):
        wid = lax.axis_index("s") * _NC + lax.axis_index("c")
        base = wid * _B_PER_W
        pltpu.sync_copy(idx_hbm.at[pl.ds(base, _B_PER_W)], idx_v)

        def body(g, carry):
            grp = idx_v[pl.ds(g * 16, 16)]
            for j in range(16):
                pltpu.make_async_copy(
                    table_hbm.at[pl.ds(grp[j], 1)],
                    rows_v.at[pl.ds(g * 16 + j, 1)],
                    sem,
                ).start()
            return carry

        lax.fori_loop(0, _B_PER_W // 16, body, 0)
        pltpu.make_async_copy(
            table_hbm.at[pl.ds(0, _B_PER_W)], rows_v, sem
        ).wait()
        pltpu.sync_copy(rows_v, out_hbm.at[pl.ds(base, _B_PER_W)])

    return gather_kernel


_gather = _make_gather()


def kernel(idx, appearance):
    return _gather(idx.astype(jnp.int32), appearance)


# speedup vs baseline: 3.8226x; 3.8226x over previous
import functools

import jax
import jax.numpy as jnp
from jax import lax
from jax.experimental import pallas as pl
from jax.experimental.pallas import tpu as pltpu
from jax.experimental.pallas import tpu_sc as plsc

NUM_SIGNALS = 100000
ROW_WIDTH = 70
BATCH = 4096

_info = plsc.get_sparse_core_info()
_NC, _NS = _info.num_cores, _info.num_subcores
_NW = _NC * _NS
_B_PER_W = BATCH // _NW


def _make_gather():
    mesh = plsc.VectorSubcoreMesh(core_axis_name="c", subcore_axis_name="s")

    @functools.partial(
        pl.kernel,
        mesh=mesh,
        out_type=jax.ShapeDtypeStruct((BATCH, ROW_WIDTH), jnp.float32),
        scratch_types=[
            pltpu.VMEM((_B_PER_W,), jnp.int32),
            pltpu.VMEM((_B_PER_W, ROW_WIDTH), jnp.float32),
            pltpu.SemaphoreType.DMA,
        ],
        compiler_params=pltpu.CompilerParams(use_tc_tiling_on_sc=True),
    )
    def gather_kernel(idx_hbm, table_hbm, out_hbm, idx_v, rows_v, sem):
        wid = lax.axis_index("s") * _NC + lax.axis_index("c")
        base = wid * _B_PER_W
        pltpu.sync_copy(idx_hbm.at[pl.ds(base, _B_PER_W)], idx_v)

        def body(g, carry):
            grp = idx_v[pl.ds(g * 16, 16)]
            for j in range(16):
                pltpu.make_async_copy(
                    table_hbm.at[pl.ds(grp[j], 1)],
                    rows_v.at[pl.ds(g * 16 + j, 1)],
                    sem,
                ).start()
            return carry

        lax.fori_loop(0, _B_PER_W // 16, body, 0)
        pltpu.make_async_copy(
            table_hbm.at[pl.ds(0, _B_PER_W)], rows_v, sem
        ).wait()
        pltpu.sync_copy(rows_v, out_hbm.at[pl.ds(base, _B_PER_W)])

    return gather_kernel


_gather = _make_gather()


def kernel(idx, appearance):
    return _gather(idx.astype(jnp.int32), appearance)


# speedup vs baseline: 5.6848x; 1.4872x over previous
import functools

import jax
import jax.numpy as jnp
from jax import lax
from jax.experimental import pallas as pl
from jax.experimental.pallas import tpu as pltpu
from jax.experimental.pallas import tpu_sc as plsc

NUM_SIGNALS = 100000
ROW_WIDTH = 70
BATCH = 4096

_info = plsc.get_sparse_core_info()
_NC, _NS = _info.num_cores, _info.num_subcores
_NW = _NC * _NS


def _make_gather():
    mesh = plsc.VectorSubcoreMesh(core_axis_name="c", subcore_axis_name="s")

    @functools.partial(
        pl.kernel,
        mesh=mesh,
        out_type=jax.ShapeDtypeStruct((ROW_WIDTH, BATCH), jnp.float32),
        scratch_types=[
            pltpu.VMEM((BATCH,), jnp.int32),
            pltpu.VMEM((NUM_SIGNALS,), jnp.float32),
            pltpu.VMEM((BATCH,), jnp.float32),
        ],
        compiler_params=pltpu.CompilerParams(
            use_tc_tiling_on_sc=True, needs_layout_passes=False
        ),
    )
    def gather_kernel(idx_hbm, table_hbm, out_hbm, idx_v, plane_v, res_v):
        wid = lax.axis_index("s") * _NC + lax.axis_index("c")
        pltpu.sync_copy(idx_hbm, idx_v)

        def do_plane(j):
            pltpu.sync_copy(table_hbm.at[j], plane_v)

            def gather_grp(k, carry):
                g = plsc.load_gather(plane_v, [idx_v[pl.ds(k * 16, 16)]])
                res_v[pl.ds(k * 16, 16)] = g
                return carry

            lax.fori_loop(0, BATCH // 16, gather_grp, 0)
            pltpu.sync_copy(res_v, out_hbm.at[j])

        do_plane(wid)
        do_plane(wid + _NW)

        @pl.when(wid < ROW_WIDTH - 2 * _NW)
        def _():
            do_plane(wid + 2 * _NW)

    return gather_kernel


_gather = _make_gather()


def kernel(idx, appearance):
    out_t = _gather(idx.astype(jnp.int32), appearance.T)
    return out_t.T
